# Initial kernel scaffold; baseline (speedup 1.0000x reference)
import functools

import jax
import jax.numpy as jnp
from jax import lax
from jax.experimental import pallas as pl
from jax.experimental.pallas import tpu as pltpu
from jax.experimental.pallas import tpu_sc as plsc

N = 10000
NP = 10240
D = 256
H = 256
E = 160000

NC = 2
NS = 16
EW = E // NS
NPASS = 2
RANGE = NP // (NC * NPASS)
SPAD = RANGE + 16
ZPT = SPAD // NS
RPT = RANGE // NS
CAP = EW + 16
BLK = 256


def _pre_linear(xp, wcat, bcat):
    def body(x_ref, w_ref, b_ref, o_ref):
        acc = lax.dot_general(x_ref[...], w_ref[...],
                              (((1,), (1,)), ((), ())),
                              preferred_element_type=jnp.float32)
        o_ref[...] = jnp.maximum(acc + b_ref[...], 0.0)

    return pl.pallas_call(
        body,
        grid=(NP // BLK,),
        in_specs=[
            pl.BlockSpec((BLK, D), lambda i: (i, 0)),
            pl.BlockSpec((2 * H, D), lambda i: (0, 0)),
            pl.BlockSpec((1, 2 * H), lambda i: (0, 0)),
        ],
        out_specs=pl.BlockSpec((BLK, 2 * H), lambda i: (i, 0)),
        out_shape=jax.ShapeDtypeStruct((NP, 2 * H), jnp.float32),
    )(xp, wcat, bcat)


def _sc_segment_sum(y, row, col, zrows):
    mesh = plsc.VectorSubcoreMesh(core_axis_name="c", subcore_axis_name="s")

    @functools.partial(
        pl.kernel,
        out_type=jax.ShapeDtypeStruct((NP, 2 * H), jnp.float32),
        mesh=mesh,
        scratch_types=[
            pltpu.VMEM((EW,), jnp.int32),
            pltpu.VMEM((EW,), jnp.int32),
            pltpu.VMEM((CAP,), jnp.int32),
            pltpu.VMEM((CAP,), jnp.int32),
            pltpu.VMEM((16, 2 * H), jnp.float32),
            pltpu.VMEM_SHARED((SPAD, 2 * H), jnp.float32),
            pltpu.SemaphoreType.DMA,
        ],
    )
    def k(y_hbm, row_hbm, col_hbm, z_hbm, out_hbm,
          row_v, col_v, ridx, cidx, rows_v, acc_sh, sem):
        cid = lax.axis_index("c")
        sid = lax.axis_index("s")
        pltpu.sync_copy(row_hbm.at[pl.ds(sid * EW, EW)], row_v)
        pltpu.sync_copy(col_hbm.at[pl.ds(sid * EW, EW)], col_v)
        for p in range(NPASS):
            base = (p * NC + cid) * RANGE
            pltpu.sync_copy(z_hbm.at[pl.ds(sid * ZPT, ZPT)],
                            acc_sh.at[pl.ds(sid * ZPT, ZPT)])
            plsc.subcore_barrier()

            def scan_body(i, cnt):
                c = col_v[pl.ds(i * 16, 16)]
                r = row_v[pl.ds(i * 16, 16)]
                m = (c >= base) & (c < base + RANGE)
                plsc.store_compressed(ridx.at[pl.ds(cnt, 16)], r, m)
                plsc.store_compressed(cidx.at[pl.ds(cnt, 16)], c - base, m)
                return cnt + jnp.sum(jnp.where(m, 1, 0))

            cnt = lax.fori_loop(0, EW // 16, scan_body, 0)
            ridx[pl.ds(cnt, 16)] = jnp.zeros((16,), jnp.int32)
            cidx[pl.ds(cnt, 16)] = jnp.full((16,), RANGE, jnp.int32)

            def gs_body(j, _):
                rvec = ridx[pl.ds(j * 16, 16)]
                cvec = cidx[pl.ds(j * 16, 16)]
                pltpu.async_copy(y_hbm.at[rvec], rows_v, sem).wait()
                pltpu.sync_copy(rows_v, acc_sh.at[cvec], add=True)
                return 0

            lax.fori_loop(0, (cnt + 15) // 16, gs_body, 0)
            plsc.subcore_barrier()
            pltpu.sync_copy(acc_sh.at[pl.ds(sid * RPT, RPT)],
                            out_hbm.at[pl.ds(base + sid * RPT, RPT)])
            plsc.subcore_barrier()

    return k(y, row, col, zrows)


def _post(xt, xp, hap, hcp,
          wih_a, bih_a, whh_a, bhh_a, wih_c, bih_c, whh_c, bhh_c,
          h2g_a, bh2g_a, h2g_c, bh2g_c, g2a_a, g2a_c, misc):
    nblk = NP // BLK

    def body(xt_ref, x_ref, ha_ref, hc_ref,
             wiha_r, biha_r, whha_r, bhha_r, wihc_r, bihc_r, whhc_r, bhhc_r,
             h2ga_r, bh2ga_r, h2gc_r, bh2gc_r, g2aa_r, g2ac_r, misc_r,
             a_ref, hna_ref, hnc_ref, v_ref, pooled):
        i = pl.program_id(0)
        x = x_ref[...]

        def mm(a, b):
            return lax.dot_general(a, b, (((1,), (1,)), ((), ())),
                                   preferred_element_type=jnp.float32)

        def gru(xt_b, h, wih, bih, whh, bhh):
            gi = mm(xt_b, wih) + bih
            gh = mm(h, whh) + bhh
            r = jax.nn.sigmoid(gi[:, :H] + gh[:, :H])
            z = jax.nn.sigmoid(gi[:, H:2 * H] + gh[:, H:2 * H])
            n = jnp.tanh(gi[:, 2 * H:] + r * gh[:, 2 * H:])
            return (1.0 - z) * n + z * h

        hn_a = gru(xt_ref[:, :H], ha_ref[...], wiha_r[...], biha_r[...],
                   whha_r[...], bhha_r[...])
        hn_c = gru(xt_ref[:, H:], hc_ref[...], wihc_r[...], bihc_r[...],
                   whhc_r[...], bhhc_r[...])
        hna_ref[...] = hn_a
        hnc_ref[...] = hn_c
        g_a = jnp.maximum(mm(hn_a, h2ga_r[...]) + bh2ga_r[...], 0.0)
        g_c = jnp.maximum(mm(hn_c, h2gc_r[...]) + bh2gc_r[...], 0.0)
        cat_a = jnp.concatenate([x, g_a], axis=1)
        cat_c = jnp.concatenate([x, g_c], axis=1)
        t = jnp.sum(cat_a * g2aa_r[...], axis=1) + misc_r[0, 0]
        a_ref[...] = jnp.maximum(t, 0.0) + jnp.log1p(jnp.exp(-jnp.abs(t)))
        valid = (i * BLK + lax.broadcasted_iota(jnp.int32, (BLK, 1), 0)) < N
        part = jnp.sum(jnp.where(valid, cat_c, 0.0), axis=0, keepdims=True)

        @pl.when(i == 0)
        def _():
            pooled[...] = part

        @pl.when(i > 0)
        def _():
            pooled[...] = pooled[...] + part

        @pl.when(i == nblk - 1)
        def _():
            v = jnp.sum(pooled[...] * g2ac_r[...]) + misc_r[0, 1]
            v_ref[...] = jnp.full((1, 128), v, jnp.float32)

    full = lambda i: (0, 0)
    blk2 = lambda i: (i, 0)
    return pl.pallas_call(
        body,
        grid=(nblk,),
        in_specs=[
            pl.BlockSpec((BLK, 2 * H), blk2),
            pl.BlockSpec((BLK, D), blk2),
            pl.BlockSpec((BLK, H), blk2),
            pl.BlockSpec((BLK, H), blk2),
            pl.BlockSpec((3 * H, H), full), pl.BlockSpec((1, 3 * H), full),
            pl.BlockSpec((3 * H, H), full), pl.BlockSpec((1, 3 * H), full),
            pl.BlockSpec((3 * H, H), full), pl.BlockSpec((1, 3 * H), full),
            pl.BlockSpec((3 * H, H), full), pl.BlockSpec((1, 3 * H), full),
            pl.BlockSpec((H, H), full), pl.BlockSpec((1, H), full),
            pl.BlockSpec((H, H), full), pl.BlockSpec((1, H), full),
            pl.BlockSpec((1, D + H), full),
            pl.BlockSpec((1, D + H), full),
            pl.BlockSpec((1, 128), full),
        ],
        out_specs=[
            pl.BlockSpec((BLK,), lambda i: (i,)),
            pl.BlockSpec((BLK, H), blk2),
            pl.BlockSpec((BLK, H), blk2),
            pl.BlockSpec((1, 128), full),
        ],
        out_shape=[
            jax.ShapeDtypeStruct((NP,), jnp.float32),
            jax.ShapeDtypeStruct((NP, H), jnp.float32),
            jax.ShapeDtypeStruct((NP, H), jnp.float32),
            jax.ShapeDtypeStruct((1, 128), jnp.float32),
        ],
        scratch_shapes=[pltpu.VMEM((1, D + H), jnp.float32)],
    )(xt, xp, hap, hcp,
      wih_a, bih_a, whh_a, bhh_a, wih_c, bih_c, whh_c, bhh_c,
      h2g_a, bh2g_a, h2g_c, bh2g_c, g2a_a, g2a_c, misc)


def kernel(x, edge_index, h_a, h_c, params, jitter):
    pa, pc = params['actor'], params['critic']
    row = edge_index[0].astype(jnp.int32)
    col = edge_index[1].astype(jnp.int32)
    pad = lambda a: jnp.pad(a, ((0, NP - N), (0, 0)))
    xp, hap, hcp = pad(x), pad(h_a), pad(h_c)

    wcat = jnp.concatenate([pa['lin1_w'], pc['lin1_w']], axis=0)
    bcat = jnp.concatenate([pa['lin1_b'], pc['lin1_b']]).reshape(1, 2 * H)
    y = _pre_linear(xp, wcat, bcat)

    zrows = jnp.zeros((SPAD, 2 * H), jnp.float32)
    xt = _sc_segment_sum(y, row, col, zrows)

    misc = jnp.zeros((1, 128), jnp.float32)
    misc = misc.at[0, 0].set(pa['g2a_b'][0]).at[0, 1].set(pc['g2a_b'][0])
    a_flat, hna, hnc, vpad = _post(
        xt, xp, hap, hcp,
        pa['w_ih'], pa['b_ih'].reshape(1, -1), pa['w_hh'], pa['b_hh'].reshape(1, -1),
        pc['w_ih'], pc['b_ih'].reshape(1, -1), pc['w_hh'], pc['b_hh'].reshape(1, -1),
        pa['h2g_w'], pa['h2g_b'].reshape(1, -1), pc['h2g_w'], pc['h2g_b'].reshape(1, -1),
        pa['g2a_w'], pc['g2a_w'], misc)

    a_probs = a_flat[:N].reshape(N, 1) + jitter
    value = vpad[0, :1]
    return a_probs, value, hna[:N], hnc[:N]


# speedup vs baseline: 4.5403x; 4.5403x over previous
import functools

import jax
import jax.numpy as jnp
from jax import lax
from jax.experimental import pallas as pl
from jax.experimental.pallas import tpu as pltpu
from jax.experimental.pallas import tpu_sc as plsc

N = 10000
NP = 10240
D = 256
H = 256
E = 160000

NC = 2
NS = 16
EW = E // NS
NQ = 4
QW = 2 * H // NQ
NPASS = NQ // NC
CHUNK = 128
NCHUNK = EW // CHUNK
REM = EW - NCHUNK * CHUNK
RPT = NP // NS
BLK = 256


def _pre_linear(xp, wcat, bcat):
    def body(x_ref, w_ref, b_ref, o_ref):
        acc = lax.dot_general(x_ref[...], w_ref[...],
                              (((1,), (1,)), ((), ())),
                              preferred_element_type=jnp.float32)
        y = jnp.maximum(acc + b_ref[...], 0.0)
        for q in range(NQ):
            o_ref[q] = y[:, q * QW:(q + 1) * QW]

    return pl.pallas_call(
        body,
        grid=(NP // BLK,),
        in_specs=[
            pl.BlockSpec((BLK, D), lambda i: (i, 0)),
            pl.BlockSpec((2 * H, D), lambda i: (0, 0)),
            pl.BlockSpec((1, 2 * H), lambda i: (0, 0)),
        ],
        out_specs=pl.BlockSpec((NQ, BLK, QW), lambda i: (0, i, 0)),
        out_shape=jax.ShapeDtypeStruct((NQ, NP, QW), jnp.float32),
    )(xp, wcat, bcat)


def _sc_segment_sum(yq, rowq, col, zrows):
    mesh = plsc.VectorSubcoreMesh(core_axis_name="c", subcore_axis_name="s")

    @functools.partial(
        pl.kernel,
        out_type=jax.ShapeDtypeStruct((NQ * NP, QW), jnp.float32),
        mesh=mesh,
        scratch_types=[
            pltpu.VMEM((CHUNK,), jnp.int32),
            pltpu.VMEM((CHUNK,), jnp.int32),
            pltpu.VMEM((CHUNK, QW), jnp.float32),
            pltpu.VMEM((CHUNK,), jnp.int32),
            pltpu.VMEM((CHUNK,), jnp.int32),
            pltpu.VMEM((CHUNK, QW), jnp.float32),
            pltpu.VMEM((REM,), jnp.int32),
            pltpu.VMEM((REM,), jnp.int32),
            pltpu.VMEM_SHARED((NP, QW), jnp.float32),
            pltpu.SemaphoreType.DMA,
            pltpu.SemaphoreType.DMA,
            pltpu.SemaphoreType.DMA,
            pltpu.SemaphoreType.DMA,
        ],
    )
    def k(y_hbm, rowq_hbm, col_hbm, z_hbm, out_hbm,
          gidx_a, cidx_a, rows_a, gidx_b, cidx_b, rows_b, gidx_r, cidx_r,
          acc_sh, sem_ia, sem_ib, sem_a, sem_b):
        cid = lax.axis_index("c")
        sid = lax.axis_index("s")
        for p in range(NPASS):
            qoff = (p * NC + cid) * NP
            qbase = (p * NC + cid) * E + sid * EW
            cbase = sid * EW

            def fetch_idx(ck, gidx, cidx, sem):
                eb = ck * CHUNK
                pltpu.async_copy(rowq_hbm.at[pl.ds(qbase + eb, CHUNK)],
                                 gidx, sem)
                pltpu.async_copy(col_hbm.at[pl.ds(cbase + eb, CHUNK)],
                                 cidx, sem)

            def wait_idx(ck, gidx, cidx, sem):
                eb = ck * CHUNK
                pltpu.make_async_copy(
                    rowq_hbm.at[pl.ds(qbase + eb, CHUNK)], gidx, sem).wait()
                pltpu.make_async_copy(
                    col_hbm.at[pl.ds(cbase + eb, CHUNK)], cidx, sem).wait()

            def start(gidx, rows, sem):
                pltpu.async_copy(y_hbm.at[gidx], rows, sem)

            def wait(gidx, rows, sem):
                pltpu.make_async_copy(y_hbm.at[gidx], rows, sem).wait()

            def scatter(rows, cidx):
                pltpu.sync_copy(rows, acc_sh.at[cidx], add=True)

            pltpu.sync_copy(z_hbm.at[pl.ds(sid * RPT, RPT)],
                            acc_sh.at[pl.ds(sid * RPT, RPT)])
            plsc.subcore_barrier()

            fetch_idx(0, gidx_a, cidx_a, sem_ia)
            fetch_idx(1, gidx_b, cidx_b, sem_ib)

            def pair_body(kk, _):
                wait_idx(2 * kk, gidx_a, cidx_a, sem_ia)
                start(gidx_a, rows_a, sem_a)
                wait_idx(2 * kk + 1, gidx_b, cidx_b, sem_ib)
                start(gidx_b, rows_b, sem_b)
                wait(gidx_a, rows_a, sem_a)
                scatter(rows_a, cidx_a)

                @pl.when(kk < NCHUNK // 2 - 1)
                def _():
                    fetch_idx(2 * kk + 2, gidx_a, cidx_a, sem_ia)

                wait(gidx_b, rows_b, sem_b)
                scatter(rows_b, cidx_b)

                @pl.when(kk < NCHUNK // 2 - 1)
                def _():
                    fetch_idx(2 * kk + 3, gidx_b, cidx_b, sem_ib)

                return 0

            lax.fori_loop(0, NCHUNK // 2, pair_body, 0)
            eb = NCHUNK * CHUNK
            pltpu.sync_copy(rowq_hbm.at[pl.ds(qbase + eb, REM)], gidx_r)
            pltpu.sync_copy(col_hbm.at[pl.ds(cbase + eb, REM)], cidx_r)
            rem_v = rows_a.at[pl.ds(0, REM)]
            pltpu.async_copy(y_hbm.at[gidx_r], rem_v, sem_a).wait()
            pltpu.sync_copy(rem_v, acc_sh.at[cidx_r], add=True)
            plsc.subcore_barrier()
            pltpu.sync_copy(acc_sh.at[pl.ds(sid * RPT, RPT)],
                            out_hbm.at[pl.ds(qoff + sid * RPT, RPT)])
            plsc.subcore_barrier()

    return k(yq, rowq, col, zrows)


def _post(xa0, xa1, xc0, xc1, xp, hap, hcp,
          wih_a, bih_a, whh_a, bhh_a, wih_c, bih_c, whh_c, bhh_c,
          h2g_a, bh2g_a, h2g_c, bh2g_c, g2a_a, g2a_c, misc):
    nblk = NP // BLK

    def body(xa0_ref, xa1_ref, xc0_ref, xc1_ref, x_ref, ha_ref, hc_ref,
             wiha_r, biha_r, whha_r, bhha_r, wihc_r, bihc_r, whhc_r, bhhc_r,
             h2ga_r, bh2ga_r, h2gc_r, bh2gc_r, g2aa_r, g2ac_r, misc_r,
             a_ref, hna_ref, hnc_ref, v_ref, pooled):
        i = pl.program_id(0)
        x = x_ref[...]
        xt_a = jnp.concatenate([xa0_ref[...], xa1_ref[...]], axis=1)
        xt_c = jnp.concatenate([xc0_ref[...], xc1_ref[...]], axis=1)

        def mm(a, b):
            return lax.dot_general(a.astype(jnp.bfloat16),
                                   b.astype(jnp.bfloat16),
                                   (((1,), (1,)), ((), ())),
                                   preferred_element_type=jnp.float32)

        def gru(xt_b, h, wih, bih, whh, bhh):
            gi = mm(xt_b, wih) + bih
            gh = mm(h, whh) + bhh
            r = jax.nn.sigmoid(gi[:, :H] + gh[:, :H])
            z = jax.nn.sigmoid(gi[:, H:2 * H] + gh[:, H:2 * H])
            n = jnp.tanh(gi[:, 2 * H:] + r * gh[:, 2 * H:])
            return (1.0 - z) * n + z * h

        hn_a = gru(xt_a, ha_ref[...], wiha_r[...], biha_r[...],
                   whha_r[...], bhha_r[...])
        hn_c = gru(xt_c, hc_ref[...], wihc_r[...], bihc_r[...],
                   whhc_r[...], bhhc_r[...])
        hna_ref[...] = hn_a
        hnc_ref[...] = hn_c
        g_a = jnp.maximum(mm(hn_a, h2ga_r[...]) + bh2ga_r[...], 0.0)
        g_c = jnp.maximum(mm(hn_c, h2gc_r[...]) + bh2gc_r[...], 0.0)
        cat_a = jnp.concatenate([x, g_a], axis=1)
        cat_c = jnp.concatenate([x, g_c], axis=1)
        t = jnp.sum(cat_a * g2aa_r[...], axis=1) + misc_r[0, 0]
        a_ref[...] = jnp.maximum(t, 0.0) + jnp.log1p(jnp.exp(-jnp.abs(t)))
        valid = (i * BLK + lax.broadcasted_iota(jnp.int32, (BLK, 1), 0)) < N
        part = jnp.sum(jnp.where(valid, cat_c, 0.0), axis=0, keepdims=True)

        @pl.when(i == 0)
        def _():
            pooled[...] = part

        @pl.when(i > 0)
        def _():
            pooled[...] = pooled[...] + part

        @pl.when(i == nblk - 1)
        def _():
            v = jnp.sum(pooled[...] * g2ac_r[...]) + misc_r[0, 1]
            v_ref[...] = jnp.full((1, 128), v, jnp.float32)

    full = lambda i: (0, 0)
    blk2 = lambda i: (i, 0)
    return pl.pallas_call(
        body,
        grid=(nblk,),
        in_specs=[
            pl.BlockSpec((BLK, QW), blk2),
            pl.BlockSpec((BLK, QW), blk2),
            pl.BlockSpec((BLK, QW), blk2),
            pl.BlockSpec((BLK, QW), blk2),
            pl.BlockSpec((BLK, D), blk2),
            pl.BlockSpec((BLK, H), blk2),
            pl.BlockSpec((BLK, H), blk2),
            pl.BlockSpec((3 * H, H), full), pl.BlockSpec((1, 3 * H), full),
            pl.BlockSpec((3 * H, H), full), pl.BlockSpec((1, 3 * H), full),
            pl.BlockSpec((3 * H, H), full), pl.BlockSpec((1, 3 * H), full),
            pl.BlockSpec((3 * H, H), full), pl.BlockSpec((1, 3 * H), full),
            pl.BlockSpec((H, H), full), pl.BlockSpec((1, H), full),
            pl.BlockSpec((H, H), full), pl.BlockSpec((1, H), full),
            pl.BlockSpec((1, D + H), full),
            pl.BlockSpec((1, D + H), full),
            pl.BlockSpec((1, 128), full),
        ],
        out_specs=[
            pl.BlockSpec((BLK,), lambda i: (i,)),
            pl.BlockSpec((BLK, H), blk2),
            pl.BlockSpec((BLK, H), blk2),
            pl.BlockSpec((1, 128), full),
        ],
        out_shape=[
            jax.ShapeDtypeStruct((NP,), jnp.float32),
            jax.ShapeDtypeStruct((NP, H), jnp.float32),
            jax.ShapeDtypeStruct((NP, H), jnp.float32),
            jax.ShapeDtypeStruct((1, 128), jnp.float32),
        ],
        scratch_shapes=[pltpu.VMEM((1, D + H), jnp.float32)],
    )(xa0, xa1, xc0, xc1, xp, hap, hcp,
      wih_a, bih_a, whh_a, bhh_a, wih_c, bih_c, whh_c, bhh_c,
      h2g_a, bh2g_a, h2g_c, bh2g_c, g2a_a, g2a_c, misc)


def kernel(x, edge_index, h_a, h_c, params, jitter):
    pa, pc = params['actor'], params['critic']
    row = edge_index[0].astype(jnp.int32)
    col = edge_index[1].astype(jnp.int32)
    pad = lambda a: jnp.pad(a, ((0, NP - N), (0, 0)))
    xp, hap, hcp = pad(x), pad(h_a), pad(h_c)

    wcat = jnp.concatenate([pa['lin1_w'], pc['lin1_w']], axis=0)
    bcat = jnp.concatenate([pa['lin1_b'], pc['lin1_b']]).reshape(1, 2 * H)
    y4 = _pre_linear(xp, wcat, bcat)

    zrows = jnp.zeros((NP, QW), jnp.float32)
    rowq = (row[None, :] + (jnp.arange(NQ, dtype=jnp.int32) * NP)[:, None]
            ).reshape(NQ * E)
    xtq = _sc_segment_sum(y4.reshape(NQ * NP, QW), rowq, col, zrows)

    misc = jnp.zeros((1, 128), jnp.float32)
    misc = misc.at[0, 0].set(pa['g2a_b'][0]).at[0, 1].set(pc['g2a_b'][0])
    a_flat, hna, hnc, vpad = _post(
        xtq[:NP], xtq[NP:2 * NP], xtq[2 * NP:3 * NP], xtq[3 * NP:],
        xp, hap, hcp,
        pa['w_ih'], pa['b_ih'].reshape(1, -1), pa['w_hh'], pa['b_hh'].reshape(1, -1),
        pc['w_ih'], pc['b_ih'].reshape(1, -1), pc['w_hh'], pc['b_hh'].reshape(1, -1),
        pa['h2g_w'], pa['h2g_b'].reshape(1, -1), pc['h2g_w'], pc['h2g_b'].reshape(1, -1),
        pa['g2a_w'], pc['g2a_w'], misc)

    a_probs = a_flat[:N].reshape(N, 1) + jitter
    value = vpad[0, :1]
    return a_probs, value, hna[:N], hnc[:N]


# speedup vs baseline: 4.5793x; 1.0086x over previous
import functools

import jax
import jax.numpy as jnp
from jax import lax
from jax.experimental import pallas as pl
from jax.experimental.pallas import tpu as pltpu
from jax.experimental.pallas import tpu_sc as plsc

N = 10000
NP = 10240
D = 256
H = 256
E = 160000

NC = 2
NS = 16
EW = E // NS
NQ = 4
QW = 2 * H // NQ
NPASS = NQ // NC
CHUNK = 64
NCHUNK = EW // CHUNK
REM = EW - NCHUNK * CHUNK
RPT = NP // NS
BLK = 256


def _pre_linear(xp, wcat, bcat):
    def body(x_ref, w_ref, b_ref, o_ref):
        acc = lax.dot_general(x_ref[...], w_ref[...],
                              (((1,), (1,)), ((), ())),
                              preferred_element_type=jnp.float32)
        y = jnp.maximum(acc + b_ref[...], 0.0)
        for q in range(NQ):
            o_ref[q] = y[:, q * QW:(q + 1) * QW]

    return pl.pallas_call(
        body,
        grid=(NP // BLK,),
        in_specs=[
            pl.BlockSpec((BLK, D), lambda i: (i, 0)),
            pl.BlockSpec((2 * H, D), lambda i: (0, 0)),
            pl.BlockSpec((1, 2 * H), lambda i: (0, 0)),
        ],
        out_specs=pl.BlockSpec((NQ, BLK, QW), lambda i: (0, i, 0)),
        out_shape=jax.ShapeDtypeStruct((NQ, NP, QW), jnp.float32),
    )(xp, wcat, bcat)


def _sc_segment_sum(yq, row, col, zrows):
    mesh = plsc.VectorSubcoreMesh(core_axis_name="c", subcore_axis_name="s")

    @functools.partial(
        pl.kernel,
        out_type=jax.ShapeDtypeStruct((NQ * NP, QW), jnp.float32),
        mesh=mesh,
        scratch_types=[
            pltpu.VMEM((EW,), jnp.int32),
            pltpu.VMEM((EW,), jnp.int32),
            pltpu.VMEM((CHUNK,), jnp.int32),
            pltpu.VMEM((CHUNK,), jnp.int32),
            pltpu.VMEM((CHUNK, QW), jnp.float32),
            pltpu.VMEM((CHUNK,), jnp.int32),
            pltpu.VMEM((CHUNK,), jnp.int32),
            pltpu.VMEM((CHUNK, QW), jnp.float32),
            pltpu.VMEM_SHARED((NP, QW), jnp.float32),
            pltpu.SemaphoreType.DMA,
            pltpu.SemaphoreType.DMA,
        ],
    )
    def k(y_hbm, row_hbm, col_hbm, z_hbm, out_hbm,
          row_v, col_v, gidx_a, cidx_a, rows_a, gidx_b, cidx_b, rows_b,
          acc_sh, sem_a, sem_b):
        cid = lax.axis_index("c")
        sid = lax.axis_index("s")
        pltpu.sync_copy(row_hbm.at[pl.ds(sid * EW, EW)], row_v)
        pltpu.sync_copy(col_hbm.at[pl.ds(sid * EW, EW)], col_v)
        for p in range(NPASS):
            qoff = (p * NC + cid) * NP

            def build(ck, gidx, cidx):
                eb = ck * CHUNK
                for v in range(CHUNK // 16):
                    gidx[pl.ds(v * 16, 16)] = (
                        row_v[pl.ds(eb + v * 16, 16)] + qoff)
                    cidx[pl.ds(v * 16, 16)] = col_v[pl.ds(eb + v * 16, 16)]

            def start(gidx, rows, sem):
                pltpu.async_copy(y_hbm.at[gidx], rows, sem)

            def wait(gidx, rows, sem):
                pltpu.make_async_copy(y_hbm.at[gidx], rows, sem).wait()

            def scatter(rows, cidx):
                pltpu.sync_copy(rows, acc_sh.at[cidx], add=True)

            pltpu.sync_copy(z_hbm.at[pl.ds(sid * RPT, RPT)],
                            acc_sh.at[pl.ds(sid * RPT, RPT)])
            plsc.subcore_barrier()

            build(0, gidx_a, cidx_a)
            start(gidx_a, rows_a, sem_a)

            def pair_body(kk, _):
                build(2 * kk + 1, gidx_b, cidx_b)
                start(gidx_b, rows_b, sem_b)
                wait(gidx_a, rows_a, sem_a)
                scatter(rows_a, cidx_a)

                @pl.when(kk < NCHUNK // 2 - 1)
                def _():
                    build(2 * kk + 2, gidx_a, cidx_a)
                    start(gidx_a, rows_a, sem_a)

                wait(gidx_b, rows_b, sem_b)
                scatter(rows_b, cidx_b)
                return 0

            lax.fori_loop(0, NCHUNK // 2, pair_body, 0)
            rr = row_v[pl.ds(NCHUNK * CHUNK, REM)] + qoff
            cc = col_v[pl.ds(NCHUNK * CHUNK, REM)]
            rem_v = rows_a.at[pl.ds(0, REM)]
            pltpu.async_copy(y_hbm.at[rr], rem_v, sem_a).wait()
            pltpu.sync_copy(rem_v, acc_sh.at[cc], add=True)
            plsc.subcore_barrier()
            pltpu.sync_copy(acc_sh.at[pl.ds(sid * RPT, RPT)],
                            out_hbm.at[pl.ds(qoff + sid * RPT, RPT)])
            plsc.subcore_barrier()

    return k(yq, row, col, zrows)


def _gh_linear(hap, hcp, whh_a, bhh_a, whh_c, bhh_c):
    def body(ha_ref, hc_ref, wa_r, ba_r, wc_r, bc_r, ga_ref, gc_ref):
        def mm(a, b):
            return lax.dot_general(a.astype(jnp.bfloat16),
                                   b.astype(jnp.bfloat16),
                                   (((1,), (1,)), ((), ())),
                                   preferred_element_type=jnp.float32)

        ga_ref[...] = mm(ha_ref[...], wa_r[...]) + ba_r[...]
        gc_ref[...] = mm(hc_ref[...], wc_r[...]) + bc_r[...]

    full = lambda i: (0, 0)
    blk2 = lambda i: (i, 0)
    return pl.pallas_call(
        body,
        grid=(NP // BLK,),
        in_specs=[
            pl.BlockSpec((BLK, H), blk2),
            pl.BlockSpec((BLK, H), blk2),
            pl.BlockSpec((3 * H, H), full), pl.BlockSpec((1, 3 * H), full),
            pl.BlockSpec((3 * H, H), full), pl.BlockSpec((1, 3 * H), full),
        ],
        out_specs=[pl.BlockSpec((BLK, 3 * H), blk2),
                   pl.BlockSpec((BLK, 3 * H), blk2)],
        out_shape=[jax.ShapeDtypeStruct((NP, 3 * H), jnp.float32),
                   jax.ShapeDtypeStruct((NP, 3 * H), jnp.float32)],
    )(hap, hcp, whh_a, bhh_a, whh_c, bhh_c)


def _post(xa0, xa1, xc0, xc1, xp, hap, hcp, gha, ghc,
          wih_a, bih_a, wih_c, bih_c,
          h2g_a, bh2g_a, h2g_c, bh2g_c, g2a_a, g2a_c, misc):
    nblk = NP // BLK

    def body(xa0_ref, xa1_ref, xc0_ref, xc1_ref, x_ref, ha_ref, hc_ref,
             gha_ref, ghc_ref,
             wiha_r, biha_r, wihc_r, bihc_r,
             h2ga_r, bh2ga_r, h2gc_r, bh2gc_r, g2aa_r, g2ac_r, misc_r,
             a_ref, hna_ref, hnc_ref, v_ref, pooled):
        i = pl.program_id(0)
        x = x_ref[...]
        xt_a = jnp.concatenate([xa0_ref[...], xa1_ref[...]], axis=1)
        xt_c = jnp.concatenate([xc0_ref[...], xc1_ref[...]], axis=1)

        def mm(a, b):
            return lax.dot_general(a.astype(jnp.bfloat16),
                                   b.astype(jnp.bfloat16),
                                   (((1,), (1,)), ((), ())),
                                   preferred_element_type=jnp.float32)

        def gru(xt_b, h, gh, wih, bih):
            gi = mm(xt_b, wih) + bih
            r = jax.nn.sigmoid(gi[:, :H] + gh[:, :H])
            z = jax.nn.sigmoid(gi[:, H:2 * H] + gh[:, H:2 * H])
            n = jnp.tanh(gi[:, 2 * H:] + r * gh[:, 2 * H:])
            return (1.0 - z) * n + z * h

        hn_a = gru(xt_a, ha_ref[...], gha_ref[...], wiha_r[...], biha_r[...])
        hn_c = gru(xt_c, hc_ref[...], ghc_ref[...], wihc_r[...], bihc_r[...])
        hna_ref[...] = hn_a
        hnc_ref[...] = hn_c
        g_a = jnp.maximum(mm(hn_a, h2ga_r[...]) + bh2ga_r[...], 0.0)
        g_c = jnp.maximum(mm(hn_c, h2gc_r[...]) + bh2gc_r[...], 0.0)
        cat_a = jnp.concatenate([x, g_a], axis=1)
        cat_c = jnp.concatenate([x, g_c], axis=1)
        t = jnp.sum(cat_a * g2aa_r[...], axis=1) + misc_r[0, 0]
        a_ref[...] = jnp.maximum(t, 0.0) + jnp.log1p(jnp.exp(-jnp.abs(t)))
        valid = (i * BLK + lax.broadcasted_iota(jnp.int32, (BLK, 1), 0)) < N
        part = jnp.sum(jnp.where(valid, cat_c, 0.0), axis=0, keepdims=True)

        @pl.when(i == 0)
        def _():
            pooled[...] = part

        @pl.when(i > 0)
        def _():
            pooled[...] = pooled[...] + part

        @pl.when(i == nblk - 1)
        def _():
            v = jnp.sum(pooled[...] * g2ac_r[...]) + misc_r[0, 1]
            v_ref[...] = jnp.full((1, 128), v, jnp.float32)

    full = lambda i: (0, 0)
    blk2 = lambda i: (i, 0)
    return pl.pallas_call(
        body,
        grid=(nblk,),
        in_specs=[
            pl.BlockSpec((BLK, QW), blk2),
            pl.BlockSpec((BLK, QW), blk2),
            pl.BlockSpec((BLK, QW), blk2),
            pl.BlockSpec((BLK, QW), blk2),
            pl.BlockSpec((BLK, D), blk2),
            pl.BlockSpec((BLK, H), blk2),
            pl.BlockSpec((BLK, H), blk2),
            pl.BlockSpec((BLK, 3 * H), blk2),
            pl.BlockSpec((BLK, 3 * H), blk2),
            pl.BlockSpec((3 * H, H), full), pl.BlockSpec((1, 3 * H), full),
            pl.BlockSpec((3 * H, H), full), pl.BlockSpec((1, 3 * H), full),
            pl.BlockSpec((H, H), full), pl.BlockSpec((1, H), full),
            pl.BlockSpec((H, H), full), pl.BlockSpec((1, H), full),
            pl.BlockSpec((1, D + H), full),
            pl.BlockSpec((1, D + H), full),
            pl.BlockSpec((1, 128), full),
        ],
        out_specs=[
            pl.BlockSpec((BLK,), lambda i: (i,)),
            pl.BlockSpec((BLK, H), blk2),
            pl.BlockSpec((BLK, H), blk2),
            pl.BlockSpec((1, 128), full),
        ],
        out_shape=[
            jax.ShapeDtypeStruct((NP,), jnp.float32),
            jax.ShapeDtypeStruct((NP, H), jnp.float32),
            jax.ShapeDtypeStruct((NP, H), jnp.float32),
            jax.ShapeDtypeStruct((1, 128), jnp.float32),
        ],
        scratch_shapes=[pltpu.VMEM((1, D + H), jnp.float32)],
    )(xa0, xa1, xc0, xc1, xp, hap, hcp, gha, ghc,
      wih_a, bih_a, wih_c, bih_c,
      h2g_a, bh2g_a, h2g_c, bh2g_c, g2a_a, g2a_c, misc)


def kernel(x, edge_index, h_a, h_c, params, jitter):
    pa, pc = params['actor'], params['critic']
    row = edge_index[0].astype(jnp.int32)
    col = edge_index[1].astype(jnp.int32)
    pad = lambda a: jnp.pad(a, ((0, NP - N), (0, 0)))
    xp, hap, hcp = pad(x), pad(h_a), pad(h_c)

    wcat = jnp.concatenate([pa['lin1_w'], pc['lin1_w']], axis=0)
    bcat = jnp.concatenate([pa['lin1_b'], pc['lin1_b']]).reshape(1, 2 * H)
    y4 = _pre_linear(xp, wcat, bcat)

    zrows = jnp.zeros((NP, QW), jnp.float32)
    xtq = _sc_segment_sum(y4.reshape(NQ * NP, QW), row, col, zrows)

    gha, ghc = _gh_linear(hap, hcp,
                          pa['w_hh'], pa['b_hh'].reshape(1, -1),
                          pc['w_hh'], pc['b_hh'].reshape(1, -1))

    misc = jnp.zeros((1, 128), jnp.float32)
    misc = misc.at[0, 0].set(pa['g2a_b'][0]).at[0, 1].set(pc['g2a_b'][0])
    a_flat, hna, hnc, vpad = _post(
        xtq[:NP], xtq[NP:2 * NP], xtq[2 * NP:3 * NP], xtq[3 * NP:],
        xp, hap, hcp, gha, ghc,
        pa['w_ih'], pa['b_ih'].reshape(1, -1),
        pc['w_ih'], pc['b_ih'].reshape(1, -1),
        pa['h2g_w'], pa['h2g_b'].reshape(1, -1), pc['h2g_w'], pc['h2g_b'].reshape(1, -1),
        pa['g2a_w'], pc['g2a_w'], misc)

    a_probs = a_flat[:N].reshape(N, 1) + jitter
    value = vpad[0, :1]
    return a_probs, value, hna[:N], hnc[:N]


# speedup vs baseline: 4.7960x; 1.0473x over previous
import functools

import jax
import jax.numpy as jnp
from jax import lax
from jax.experimental import pallas as pl
from jax.experimental.pallas import tpu as pltpu
from jax.experimental.pallas import tpu_sc as plsc

N = 10000
NP = 10240
D = 256
H = 256
E = 160000

NC = 2
NS = 16
EW = E // NS
NQ = 4
QW = 2 * H // NQ
NPASS = NQ // NC
CHUNK = 64
NCHUNK = EW // CHUNK
REM = EW - NCHUNK * CHUNK
RPT = NP // NS
BLK = 256


def _pre_linear(xp, wcat, bcat):
    def body(x_ref, w_ref, b_ref, o_ref):
        acc = lax.dot_general(x_ref[...], w_ref[...],
                              (((1,), (1,)), ((), ())),
                              preferred_element_type=jnp.float32)
        y = jnp.maximum(acc + b_ref[...], 0.0)
        for q in range(NQ):
            o_ref[q] = y[:, q * QW:(q + 1) * QW]

    return pl.pallas_call(
        body,
        grid=(NP // BLK,),
        in_specs=[
            pl.BlockSpec((BLK, D), lambda i: (i, 0)),
            pl.BlockSpec((2 * H, D), lambda i: (0, 0)),
            pl.BlockSpec((1, 2 * H), lambda i: (0, 0)),
        ],
        out_specs=pl.BlockSpec((NQ, BLK, QW), lambda i: (0, i, 0)),
        out_shape=jax.ShapeDtypeStruct((NQ, NP, QW), jnp.float32),
    )(xp, wcat, bcat)


def _sc_segment_sum(yq, row, col, zrows):
    mesh = plsc.VectorSubcoreMesh(core_axis_name="c", subcore_axis_name="s")

    @functools.partial(
        pl.kernel,
        out_type=jax.ShapeDtypeStruct((NQ * NP, QW), jnp.float32),
        mesh=mesh,
        scratch_types=[
            pltpu.VMEM((EW,), jnp.int32),
            pltpu.VMEM((EW,), jnp.int32),
            pltpu.VMEM((CHUNK,), jnp.int32),
            pltpu.VMEM((CHUNK,), jnp.int32),
            pltpu.VMEM((CHUNK, QW), jnp.float32),
            pltpu.VMEM((CHUNK,), jnp.int32),
            pltpu.VMEM((CHUNK,), jnp.int32),
            pltpu.VMEM((CHUNK, QW), jnp.float32),
            pltpu.VMEM_SHARED((NP, QW), jnp.float32),
            pltpu.SemaphoreType.DMA,
            pltpu.SemaphoreType.DMA,
        ],
    )
    def k(y_hbm, row_hbm, col_hbm, z_hbm, out_hbm,
          row_v, col_v, gidx_a, cidx_a, rows_a, gidx_b, cidx_b, rows_b,
          acc_sh, sem_a, sem_b):
        cid = lax.axis_index("c")
        sid = lax.axis_index("s")
        pltpu.sync_copy(row_hbm.at[pl.ds(sid * EW, EW)], row_v)
        pltpu.sync_copy(col_hbm.at[pl.ds(sid * EW, EW)], col_v)
        for p in range(NPASS):
            qoff = (p * NC + cid) * NP

            def build(ck, gidx, cidx):
                eb = ck * CHUNK
                for v in range(CHUNK // 16):
                    gidx[pl.ds(v * 16, 16)] = (
                        row_v[pl.ds(eb + v * 16, 16)] + qoff)
                    cidx[pl.ds(v * 16, 16)] = col_v[pl.ds(eb + v * 16, 16)]

            def start(gidx, rows, sem):
                pltpu.async_copy(y_hbm.at[gidx], rows, sem)

            def wait(gidx, rows, sem):
                pltpu.make_async_copy(y_hbm.at[gidx], rows, sem).wait()

            def scatter(rows, cidx):
                pltpu.sync_copy(rows, acc_sh.at[cidx], add=True)

            pltpu.sync_copy(z_hbm.at[pl.ds(sid * RPT, RPT)],
                            acc_sh.at[pl.ds(sid * RPT, RPT)])
            plsc.subcore_barrier()

            build(0, gidx_a, cidx_a)
            start(gidx_a, rows_a, sem_a)

            def pair_body(kk, _):
                build(2 * kk + 1, gidx_b, cidx_b)
                start(gidx_b, rows_b, sem_b)
                wait(gidx_a, rows_a, sem_a)
                scatter(rows_a, cidx_a)

                @pl.when(kk < NCHUNK // 2 - 1)
                def _():
                    build(2 * kk + 2, gidx_a, cidx_a)
                    start(gidx_a, rows_a, sem_a)

                wait(gidx_b, rows_b, sem_b)
                scatter(rows_b, cidx_b)
                return 0

            lax.fori_loop(0, NCHUNK // 2, pair_body, 0)
            rr = row_v[pl.ds(NCHUNK * CHUNK, REM)] + qoff
            cc = col_v[pl.ds(NCHUNK * CHUNK, REM)]
            rem_v = rows_a.at[pl.ds(0, REM)]
            pltpu.async_copy(y_hbm.at[rr], rem_v, sem_a).wait()
            pltpu.sync_copy(rem_v, acc_sh.at[cc], add=True)
            plsc.subcore_barrier()
            pltpu.sync_copy(acc_sh.at[pl.ds(sid * RPT, RPT)],
                            out_hbm.at[pl.ds(qoff + sid * RPT, RPT)])
            plsc.subcore_barrier()

    return k(yq, row, col, zrows)


def _post(xa0, xa1, xc0, xc1, xp, hap, hcp,
          wih_a, bih_a, whh_a, bhh_a, wih_c, bih_c, whh_c, bhh_c,
          h2g_a, bh2g_a, h2g_c, bh2g_c, g2a_a, g2a_c, misc):
    nblk = NP // BLK

    def body(xa0_ref, xa1_ref, xc0_ref, xc1_ref, x_ref, ha_ref, hc_ref,
             wiha_r, biha_r, whha_r, bhha_r, wihc_r, bihc_r, whhc_r, bhhc_r,
             h2ga_r, bh2ga_r, h2gc_r, bh2gc_r, g2aa_r, g2ac_r, misc_r,
             a_ref, hna_ref, hnc_ref, v_ref, pooled):
        i = pl.program_id(0)
        x = x_ref[...]
        xt_a = jnp.concatenate([xa0_ref[...], xa1_ref[...]], axis=1)
        xt_c = jnp.concatenate([xc0_ref[...], xc1_ref[...]], axis=1)

        def mm(a, b):
            return lax.dot_general(a.astype(jnp.bfloat16),
                                   b.astype(jnp.bfloat16),
                                   (((1,), (1,)), ((), ())),
                                   preferred_element_type=jnp.float32)

        def gru(xt_b, h, wih, bih, whh, bhh):
            gi = mm(xt_b, wih) + bih
            gh = mm(h, whh) + bhh
            r = jax.nn.sigmoid(gi[:, :H] + gh[:, :H])
            z = jax.nn.sigmoid(gi[:, H:2 * H] + gh[:, H:2 * H])
            n = jnp.tanh(gi[:, 2 * H:] + r * gh[:, 2 * H:])
            return (1.0 - z) * n + z * h

        hn_a = gru(xt_a, ha_ref[...], wiha_r[...], biha_r[...],
                   whha_r[...], bhha_r[...])
        hn_c = gru(xt_c, hc_ref[...], wihc_r[...], bihc_r[...],
                   whhc_r[...], bhhc_r[...])
        hna_ref[...] = hn_a
        hnc_ref[...] = hn_c
        g_a = jnp.maximum(mm(hn_a, h2ga_r[...]) + bh2ga_r[...], 0.0)
        g_c = jnp.maximum(mm(hn_c, h2gc_r[...]) + bh2gc_r[...], 0.0)
        cat_a = jnp.concatenate([x, g_a], axis=1)
        cat_c = jnp.concatenate([x, g_c], axis=1)
        t = jnp.sum(cat_a * g2aa_r[...], axis=1) + misc_r[0, 0]
        a_ref[...] = jnp.maximum(t, 0.0) + jnp.log1p(jnp.exp(-jnp.abs(t)))
        valid = (i * BLK + lax.broadcasted_iota(jnp.int32, (BLK, 1), 0)) < N
        part = jnp.sum(jnp.where(valid, cat_c, 0.0), axis=0, keepdims=True)

        @pl.when(i == 0)
        def _():
            pooled[...] = part

        @pl.when(i > 0)
        def _():
            pooled[...] = pooled[...] + part

        @pl.when(i == nblk - 1)
        def _():
            v = jnp.sum(pooled[...] * g2ac_r[...]) + misc_r[0, 1]
            v_ref[...] = jnp.full((1, 128), v, jnp.float32)

    full = lambda i: (0, 0)
    blk2 = lambda i: (i, 0)
    return pl.pallas_call(
        body,
        grid=(nblk,),
        in_specs=[
            pl.BlockSpec((BLK, QW), blk2),
            pl.BlockSpec((BLK, QW), blk2),
            pl.BlockSpec((BLK, QW), blk2),
            pl.BlockSpec((BLK, QW), blk2),
            pl.BlockSpec((BLK, D), blk2),
            pl.BlockSpec((BLK, H), blk2),
            pl.BlockSpec((BLK, H), blk2),
            pl.BlockSpec((3 * H, H), full), pl.BlockSpec((1, 3 * H), full),
            pl.BlockSpec((3 * H, H), full), pl.BlockSpec((1, 3 * H), full),
            pl.BlockSpec((3 * H, H), full), pl.BlockSpec((1, 3 * H), full),
            pl.BlockSpec((3 * H, H), full), pl.BlockSpec((1, 3 * H), full),
            pl.BlockSpec((H, H), full), pl.BlockSpec((1, H), full),
            pl.BlockSpec((H, H), full), pl.BlockSpec((1, H), full),
            pl.BlockSpec((1, D + H), full),
            pl.BlockSpec((1, D + H), full),
            pl.BlockSpec((1, 128), full),
        ],
        out_specs=[
            pl.BlockSpec((BLK,), lambda i: (i,)),
            pl.BlockSpec((BLK, H), blk2),
            pl.BlockSpec((BLK, H), blk2),
            pl.BlockSpec((1, 128), full),
        ],
        out_shape=[
            jax.ShapeDtypeStruct((NP,), jnp.float32),
            jax.ShapeDtypeStruct((NP, H), jnp.float32),
            jax.ShapeDtypeStruct((NP, H), jnp.float32),
            jax.ShapeDtypeStruct((1, 128), jnp.float32),
        ],
        scratch_shapes=[pltpu.VMEM((1, D + H), jnp.float32)],
    )(xa0, xa1, xc0, xc1, xp, hap, hcp,
      wih_a, bih_a, whh_a, bhh_a, wih_c, bih_c, whh_c, bhh_c,
      h2g_a, bh2g_a, h2g_c, bh2g_c, g2a_a, g2a_c, misc)


def kernel(x, edge_index, h_a, h_c, params, jitter):
    pa, pc = params['actor'], params['critic']
    row = edge_index[0].astype(jnp.int32)
    col = edge_index[1].astype(jnp.int32)
    pad = lambda a: jnp.pad(a, ((0, NP - N), (0, 0)))
    xp, hap, hcp = pad(x), pad(h_a), pad(h_c)

    wcat = jnp.concatenate([pa['lin1_w'], pc['lin1_w']], axis=0)
    bcat = jnp.concatenate([pa['lin1_b'], pc['lin1_b']]).reshape(1, 2 * H)
    y4 = _pre_linear(xp, wcat, bcat)

    zrows = jnp.zeros((NP, QW), jnp.float32)
    xtq = _sc_segment_sum(y4.reshape(NQ * NP, QW), row, col, zrows)

    misc = jnp.zeros((1, 128), jnp.float32)
    misc = misc.at[0, 0].set(pa['g2a_b'][0]).at[0, 1].set(pc['g2a_b'][0])
    a_flat, hna, hnc, vpad = _post(
        xtq[:NP], xtq[NP:2 * NP], xtq[2 * NP:3 * NP], xtq[3 * NP:],
        xp, hap, hcp,
        pa['w_ih'], pa['b_ih'].reshape(1, -1), pa['w_hh'], pa['b_hh'].reshape(1, -1),
        pc['w_ih'], pc['b_ih'].reshape(1, -1), pc['w_hh'], pc['b_hh'].reshape(1, -1),
        pa['h2g_w'], pa['h2g_b'].reshape(1, -1), pc['h2g_w'], pc['h2g_b'].reshape(1, -1),
        pa['g2a_w'], pc['g2a_w'], misc)

    a_probs = a_flat[:N].reshape(N, 1) + jitter
    value = vpad[0, :1]
    return a_probs, value, hna[:N], hnc[:N]
